# baseline (device time: 81472 ns/iter reference)
import jax
import jax.numpy as jnp
from jax import lax
from jax.experimental import pallas as pl
from jax.experimental.pallas import tpu as pltpu

N_DEV = 32
HL = 4
DH = 64
B = 2
SQ = 256
SKV = 256
DMODEL = 512
ROWS = B * SQ
C = ROWS // N_DEV


JB = SKV // N_DEV
HQ = 128


def _body(x_ref, wq_ref, k_hbm, v_hbm, wo_ref, out_ref,
          kvj_ref, st_ref, kvr_ref, p_ref, pb_ref, rs_ref, red_ref,
          ag_ref, kv_sems, es, er, s1, r1, s2, r2):
    my = lax.axis_index("i")

    kv_copies = []
    for t, src in enumerate((k_hbm, v_hbm)):
        cp = pltpu.make_async_copy(
            src.at[:, pl.ds(my * JB, JB), :, :], kvj_ref.at[t], kv_sems.at[t])
        cp.start()
        kv_copies.append(cp)

    bar = pltpu.get_barrier_semaphore()
    for j in range(N_DEV):
        @pl.when(j != my)
        def _():
            pl.semaphore_signal(
                bar, inc=1, device_id=j,
                device_id_type=pl.DeviceIdType.LOGICAL,
            )
    pl.semaphore_wait(bar, N_DEV - 1)

    q = jnp.dot(
        x_ref[:, :].astype(jnp.bfloat16),
        wq_ref[:, :].astype(jnp.bfloat16),
        preferred_element_type=jnp.float32,
    )

    for cp in kv_copies:
        cp.wait()

    for hg in range(HQ):
        st_ref[:, hg * DH:(hg + 1) * DH] = (
            kvj_ref[:, :, :, hg, :].reshape(2 * B * JB, DH).astype(jnp.bfloat16)
        )

    for p in range(N_DEV):
        @pl.when(p != my)
        def _():
            pltpu.make_async_remote_copy(
                src_ref=st_ref.at[:, pl.ds(p * HL * DH, HL * DH)],
                dst_ref=kvr_ref.at[my],
                send_sem=es.at[p],
                recv_sem=er.at[my],
                device_id=p,
                device_id_type=pl.DeviceIdType.LOGICAL,
            ).start()

    self_cp = pltpu.make_async_copy(
        st_ref.at[:, pl.ds(my * HL * DH, HL * DH)],
        kvr_ref.at[my],
        kv_sems.at[0],
    )
    self_cp.start()
    self_cp.wait()

    for p in range(N_DEV):
        @pl.when(p != my)
        def _():
            pltpu.make_async_remote_copy(
                src_ref=st_ref.at[:, pl.ds(0, HL * DH)],
                dst_ref=kvr_ref.at[p],
                send_sem=es.at[p],
                recv_sem=er.at[p],
                device_id=p,
                device_id_type=pl.DeviceIdType.LOGICAL,
            ).wait_recv()

    for p in range(N_DEV):
        @pl.when(p != my)
        def _():
            pltpu.make_async_remote_copy(
                src_ref=st_ref.at[:, pl.ds(p * HL * DH, HL * DH)],
                dst_ref=kvr_ref.at[my],
                send_sem=es.at[p],
                recv_sem=er.at[p],
                device_id=p,
                device_id_type=pl.DeviceIdType.LOGICAL,
            ).wait_send()

    ri = lax.broadcasted_iota(jnp.int32, (SQ, SKV), 0) // 64
    ci = lax.broadcasted_iota(jnp.int32, (SQ, SKV), 1) // 64
    mask = (ri == ci) | (ci == 0) | (((ri + ci) % 3) == 0)

    for b in range(B):
        acc = None
        for h in range(HL):
            qh = q[b * SQ:(b + 1) * SQ, h * DH:(h + 1) * DH].astype(jnp.bfloat16)
            kh = kvr_ref[:, b * JB:(b + 1) * JB,
                         h * DH:(h + 1) * DH].reshape(SKV, DH)
            s = lax.dot_general(
                qh, kh, (((1,), (1,)), ((), ())),
                preferred_element_type=jnp.float32,
            ) * 0.125
            s = jnp.where(mask, s, -1e9)
            m = jnp.max(s, axis=1, keepdims=True)
            w = jnp.exp(s - m)
            w = w / jnp.sum(w, axis=1, keepdims=True)
            vh = kvr_ref[:, B * JB + b * JB:B * JB + (b + 1) * JB,
                         h * DH:(h + 1) * DH].reshape(SKV, DH)
            ctx = jnp.dot(w.astype(jnp.bfloat16), vh,
                          preferred_element_type=jnp.float32)
            woh = wo_ref[h * DH:(h + 1) * DH, :].astype(jnp.bfloat16)
            pb = jnp.dot(ctx.astype(jnp.bfloat16), woh,
                         preferred_element_type=jnp.float32)
            acc = pb if acc is None else acc + pb
        p_ref[b * SQ:(b + 1) * SQ, :] = acc
    pb_ref[:, :] = p_ref[:, :].astype(jnp.bfloat16)

    for j in range(N_DEV):
        @pl.when(j != my)
        def _():
            rdma = pltpu.make_async_remote_copy(
                src_ref=pb_ref.at[pl.ds(j * C, C), :],
                dst_ref=rs_ref.at[pl.ds(my * C, C), :],
                send_sem=s1.at[j],
                recv_sem=r1.at[my],
                device_id=j,
                device_id_type=pl.DeviceIdType.LOGICAL,
            )
            rdma.start()

    rs_ref[pl.ds(my * C, C), :] = pb_ref[pl.ds(my * C, C), :]

    for j in range(N_DEV):
        @pl.when(j != my)
        def _():
            rd = pltpu.make_async_remote_copy(
                src_ref=pb_ref.at[pl.ds(0, C), :],
                dst_ref=rs_ref.at[pl.ds(j * C, C), :],
                send_sem=s1.at[j],
                recv_sem=r1.at[j],
                device_id=j,
                device_id_type=pl.DeviceIdType.LOGICAL,
            )
            rd.wait_recv()

    for j in range(N_DEV):
        @pl.when(j != my)
        def _():
            pltpu.make_async_remote_copy(
                src_ref=pb_ref.at[pl.ds(j * C, C), :],
                dst_ref=rs_ref.at[pl.ds(j * C, C), :],
                send_sem=s1.at[j],
                recv_sem=r1.at[j],
                device_id=j,
                device_id_type=pl.DeviceIdType.LOGICAL,
            ).wait_send()

    red = rs_ref[0:C, :].astype(jnp.float32)
    for j in range(1, N_DEV):
        red = red + rs_ref[j * C:(j + 1) * C, :].astype(jnp.float32)
    red_ref[:, :] = red.astype(jnp.bfloat16)
    ag_ref[pl.ds(my * C, C), :] = red_ref[:, :]

    for k in range(N_DEV):
        @pl.when(k != my)
        def _():
            rdma = pltpu.make_async_remote_copy(
                src_ref=red_ref.at[:, :],
                dst_ref=ag_ref.at[pl.ds(my * C, C), :],
                send_sem=s2.at[k],
                recv_sem=r2.at[my],
                device_id=k,
                device_id_type=pl.DeviceIdType.LOGICAL,
            )
            rdma.start()

    for k in range(N_DEV):
        @pl.when(k != my)
        def _():
            rd = pltpu.make_async_remote_copy(
                src_ref=red_ref.at[:, :],
                dst_ref=ag_ref.at[pl.ds(k * C, C), :],
                send_sem=s2.at[k],
                recv_sem=r2.at[k],
                device_id=k,
                device_id_type=pl.DeviceIdType.LOGICAL,
            )
            rd.wait_recv()

    out_ref[:, :] = ag_ref[:, :].astype(jnp.float32)

    for k in range(N_DEV):
        @pl.when(k != my)
        def _():
            pltpu.make_async_remote_copy(
                src_ref=red_ref.at[:, :],
                dst_ref=ag_ref.at[pl.ds(my * C, C), :],
                send_sem=s2.at[k],
                recv_sem=r2.at[k],
                device_id=k,
                device_id_type=pl.DeviceIdType.LOGICAL,
            ).wait_send()


def kernel(x, Wq, K_ext, V_ext, Wo):
    out = pl.pallas_call(
        _body,
        out_shape=jax.ShapeDtypeStruct((ROWS, DMODEL), jnp.float32),
        in_specs=[
            pl.BlockSpec(memory_space=pltpu.VMEM),
            pl.BlockSpec(memory_space=pltpu.VMEM),
            pl.BlockSpec(memory_space=pl.ANY),
            pl.BlockSpec(memory_space=pl.ANY),
            pl.BlockSpec(memory_space=pltpu.VMEM),
        ],
        out_specs=pl.BlockSpec(memory_space=pltpu.VMEM),
        scratch_shapes=[
            pltpu.VMEM((2, B, JB, HQ, DH), jnp.float32),
            pltpu.VMEM((2 * B * JB, HQ * DH), jnp.bfloat16),
            pltpu.VMEM((N_DEV, 2 * B * JB, HL * DH), jnp.bfloat16),
            pltpu.VMEM((ROWS, DMODEL), jnp.float32),
            pltpu.VMEM((ROWS, DMODEL), jnp.bfloat16),
            pltpu.VMEM((ROWS, DMODEL), jnp.bfloat16),
            pltpu.VMEM((C, DMODEL), jnp.bfloat16),
            pltpu.VMEM((ROWS, DMODEL), jnp.bfloat16),
            pltpu.SemaphoreType.DMA((2,)),
            pltpu.SemaphoreType.DMA((N_DEV,)),
            pltpu.SemaphoreType.DMA((N_DEV,)),
            pltpu.SemaphoreType.DMA((N_DEV,)),
            pltpu.SemaphoreType.DMA((N_DEV,)),
            pltpu.SemaphoreType.DMA((N_DEV,)),
            pltpu.SemaphoreType.DMA((N_DEV,)),
        ],
        compiler_params=pltpu.CompilerParams(collective_id=0),
    )(x.reshape(ROWS, DMODEL), Wq, K_ext, V_ext, Wo)
    return out.reshape(B, SQ, DMODEL)


# device time: 74301 ns/iter; 1.0965x vs baseline; 1.0965x over previous
import jax
import jax.numpy as jnp
from jax import lax
from jax.experimental import pallas as pl
from jax.experimental.pallas import tpu as pltpu

N_DEV = 32
HL = 4
DH = 64
B = 2
SQ = 256
SKV = 256
DMODEL = 512
ROWS = B * SQ
C = ROWS // N_DEV


def _body(x_ref, wq_ref, k_hbm, v_hbm, wo_ref, out_ref,
          k4_ref, v4_ref, kt_ref, vt_ref, p_ref, pb_ref, rs_ref, red_ref,
          ag_ref, kv_sems, tr_sems, s1, r1, s2, r2):
    my = lax.axis_index("i")

    kv_copies = []
    for t, (src, dst) in enumerate(((k_hbm, k4_ref), (v_hbm, v4_ref))):
        cp = pltpu.make_async_copy(
            src.at[:, :, pl.ds(my * HL, HL), :], dst, kv_sems.at[t])
        cp.start()
        kv_copies.append(cp)

    bar = pltpu.get_barrier_semaphore()
    for j in range(N_DEV):
        @pl.when(j != my)
        def _():
            pl.semaphore_signal(
                bar, inc=1, device_id=j,
                device_id_type=pl.DeviceIdType.LOGICAL,
            )
    pl.semaphore_wait(bar, N_DEV - 1)

    q = jnp.dot(
        x_ref[:, :].astype(jnp.bfloat16),
        wq_ref[:, :].astype(jnp.bfloat16),
        preferred_element_type=jnp.float32,
    )

    for cp in kv_copies:
        cp.wait()

    tr_copies = []
    for t, (src, dst) in enumerate(((k4_ref, kt_ref), (v4_ref, vt_ref))):
        for b in range(B):
            for h in range(HL):
                cp = pltpu.make_async_copy(
                    src.at[b, :, h, :], dst.at[b * HL + h],
                    tr_sems.at[t * B * HL + b * HL + h])
                cp.start()
                tr_copies.append(cp)
    for cp in tr_copies:
        cp.wait()

    ri = lax.broadcasted_iota(jnp.int32, (SQ, SKV), 0) // 64
    ci = lax.broadcasted_iota(jnp.int32, (SQ, SKV), 1) // 64
    mask = (ri == ci) | (ci == 0) | (((ri + ci) % 3) == 0)

    for b in range(B):
        acc = None
        for h in range(HL):
            qh = q[b * SQ:(b + 1) * SQ, h * DH:(h + 1) * DH].astype(jnp.bfloat16)
            kh = kt_ref[b * HL + h].astype(jnp.bfloat16)
            s = lax.dot_general(
                qh, kh, (((1,), (1,)), ((), ())),
                preferred_element_type=jnp.float32,
            ) * 0.125
            s = jnp.where(mask, s, -1e9)
            m = jnp.max(s, axis=1, keepdims=True)
            w = jnp.exp(s - m)
            w = w / jnp.sum(w, axis=1, keepdims=True)
            vh = vt_ref[b * HL + h].astype(jnp.bfloat16)
            ctx = jnp.dot(w.astype(jnp.bfloat16), vh,
                          preferred_element_type=jnp.float32)
            woh = wo_ref[h * DH:(h + 1) * DH, :].astype(jnp.bfloat16)
            pb = jnp.dot(ctx.astype(jnp.bfloat16), woh,
                         preferred_element_type=jnp.float32)
            acc = pb if acc is None else acc + pb
        p_ref[b * SQ:(b + 1) * SQ, :] = acc
        pb_ref[b * SQ:(b + 1) * SQ, :] = acc.astype(jnp.bfloat16)

        for j in range(b * SQ // C, (b + 1) * SQ // C):
            @pl.when(j != my)
            def _():
                rdma = pltpu.make_async_remote_copy(
                    src_ref=pb_ref.at[pl.ds(j * C, C), :],
                    dst_ref=rs_ref.at[pl.ds(my * C, C), :],
                    send_sem=s1.at[j],
                    recv_sem=r1.at[my],
                    device_id=j,
                    device_id_type=pl.DeviceIdType.LOGICAL,
                )
                rdma.start()

    rs_ref[pl.ds(my * C, C), :] = pb_ref[pl.ds(my * C, C), :]

    for j in range(N_DEV):
        @pl.when(j != my)
        def _():
            rd = pltpu.make_async_remote_copy(
                src_ref=pb_ref.at[pl.ds(0, C), :],
                dst_ref=rs_ref.at[pl.ds(j * C, C), :],
                send_sem=s1.at[j],
                recv_sem=r1.at[j],
                device_id=j,
                device_id_type=pl.DeviceIdType.LOGICAL,
            )
            rd.wait_recv()

    for j in range(N_DEV):
        @pl.when(j != my)
        def _():
            pltpu.make_async_remote_copy(
                src_ref=pb_ref.at[pl.ds(j * C, C), :],
                dst_ref=rs_ref.at[pl.ds(j * C, C), :],
                send_sem=s1.at[j],
                recv_sem=r1.at[j],
                device_id=j,
                device_id_type=pl.DeviceIdType.LOGICAL,
            ).wait_send()

    red = rs_ref[0:C, :].astype(jnp.float32)
    for j in range(1, N_DEV):
        red = red + rs_ref[j * C:(j + 1) * C, :].astype(jnp.float32)
    red_ref[:, :] = red.astype(jnp.bfloat16)
    ag_ref[pl.ds(my * C, C), :] = red_ref[:, :]

    for k in range(N_DEV):
        @pl.when(k != my)
        def _():
            rdma = pltpu.make_async_remote_copy(
                src_ref=red_ref.at[:, :],
                dst_ref=ag_ref.at[pl.ds(my * C, C), :],
                send_sem=s2.at[k],
                recv_sem=r2.at[my],
                device_id=k,
                device_id_type=pl.DeviceIdType.LOGICAL,
            )
            rdma.start()

    for k in range(N_DEV):
        @pl.when(k != my)
        def _():
            rd = pltpu.make_async_remote_copy(
                src_ref=red_ref.at[:, :],
                dst_ref=ag_ref.at[pl.ds(k * C, C), :],
                send_sem=s2.at[k],
                recv_sem=r2.at[k],
                device_id=k,
                device_id_type=pl.DeviceIdType.LOGICAL,
            )
            rd.wait_recv()

    out_ref[:, :] = ag_ref[:, :].astype(jnp.float32)

    for k in range(N_DEV):
        @pl.when(k != my)
        def _():
            pltpu.make_async_remote_copy(
                src_ref=red_ref.at[:, :],
                dst_ref=ag_ref.at[pl.ds(my * C, C), :],
                send_sem=s2.at[k],
                recv_sem=r2.at[k],
                device_id=k,
                device_id_type=pl.DeviceIdType.LOGICAL,
            ).wait_send()


def kernel(x, Wq, K_ext, V_ext, Wo):
    out = pl.pallas_call(
        _body,
        out_shape=jax.ShapeDtypeStruct((ROWS, DMODEL), jnp.float32),
        in_specs=[
            pl.BlockSpec(memory_space=pltpu.VMEM),
            pl.BlockSpec(memory_space=pltpu.VMEM),
            pl.BlockSpec(memory_space=pl.ANY),
            pl.BlockSpec(memory_space=pl.ANY),
            pl.BlockSpec(memory_space=pltpu.VMEM),
        ],
        out_specs=pl.BlockSpec(memory_space=pltpu.VMEM),
        scratch_shapes=[
            pltpu.VMEM((B, SKV, HL, DH), jnp.float32),
            pltpu.VMEM((B, SKV, HL, DH), jnp.float32),
            pltpu.VMEM((B * HL, SKV, DH), jnp.float32),
            pltpu.VMEM((B * HL, SKV, DH), jnp.float32),
            pltpu.VMEM((ROWS, DMODEL), jnp.float32),
            pltpu.VMEM((ROWS, DMODEL), jnp.bfloat16),
            pltpu.VMEM((ROWS, DMODEL), jnp.bfloat16),
            pltpu.VMEM((C, DMODEL), jnp.bfloat16),
            pltpu.VMEM((ROWS, DMODEL), jnp.bfloat16),
            pltpu.SemaphoreType.DMA((2,)),
            pltpu.SemaphoreType.DMA((2 * B * HL,)),
            pltpu.SemaphoreType.DMA((N_DEV,)),
            pltpu.SemaphoreType.DMA((N_DEV,)),
            pltpu.SemaphoreType.DMA((N_DEV,)),
            pltpu.SemaphoreType.DMA((N_DEV,)),
        ],
        compiler_params=pltpu.CompilerParams(collective_id=0),
    )(x.reshape(ROWS, DMODEL), Wq, K_ext, V_ext, Wo)
    return out.reshape(B, SQ, DMODEL)


# device time: 69563 ns/iter; 1.1712x vs baseline; 1.0681x over previous
import jax
import jax.numpy as jnp
from jax import lax
from jax.experimental import pallas as pl
from jax.experimental.pallas import tpu as pltpu

N_DEV = 32
HL = 4
DH = 64
B = 2
SQ = 256
SKV = 256
DMODEL = 512
ROWS = B * SQ
C = ROWS // N_DEV


def _body(x_ref, wq_ref, k_hbm, v_hbm, wo_ref, out_ref,
          k4_ref, v4_ref, kt_ref, vt_ref, p_ref, pb_ref, rs_ref, red_ref,
          ag_ref, kv_sems, tr_sems, s1, r1, s2, r2):
    my = lax.axis_index("i")

    kv_copies = {}
    for b in range(B):
        for t, (src, dst) in enumerate(((k_hbm, k4_ref), (v_hbm, v4_ref))):
            cp = pltpu.make_async_copy(
                src.at[b, :, pl.ds(my * HL, HL), :], dst.at[b],
                kv_sems.at[b * 2 + t])
            cp.start()
            kv_copies[(b, t)] = cp

    bar = pltpu.get_barrier_semaphore()
    for j in range(N_DEV):
        @pl.when(j != my)
        def _():
            pl.semaphore_signal(
                bar, inc=1, device_id=j,
                device_id_type=pl.DeviceIdType.LOGICAL,
            )
    pl.semaphore_wait(bar, N_DEV - 1)

    q = jnp.dot(
        x_ref[:, :].astype(jnp.bfloat16),
        wq_ref[:, :].astype(jnp.bfloat16),
        preferred_element_type=jnp.float32,
    )

    ri = lax.broadcasted_iota(jnp.int32, (SQ, SKV), 0) // 64
    ci = lax.broadcasted_iota(jnp.int32, (SQ, SKV), 1) // 64
    mask = (ri == ci) | (ci == 0) | (((ri + ci) % 3) == 0)

    for b in range(B):
        for t in range(2):
            kv_copies[(b, t)].wait()
        tr_copies = []
        for t, (src, dst) in enumerate(((k4_ref, kt_ref), (v4_ref, vt_ref))):
            for h in range(HL):
                cp = pltpu.make_async_copy(
                    src.at[b, :, h, :], dst.at[b * HL + h],
                    tr_sems.at[t * B * HL + b * HL + h])
                cp.start()
                tr_copies.append(cp)
        for cp in tr_copies:
            cp.wait()

        acc = None
        for h in range(HL):
            qh = q[b * SQ:(b + 1) * SQ, h * DH:(h + 1) * DH].astype(jnp.bfloat16)
            kh = kt_ref[b * HL + h].astype(jnp.bfloat16)
            s = lax.dot_general(
                qh, kh, (((1,), (1,)), ((), ())),
                preferred_element_type=jnp.float32,
            ) * 0.125
            s = jnp.where(mask, s, -1e9)
            m = jnp.max(s, axis=1, keepdims=True)
            w = jnp.exp(s - m)
            w = w / jnp.sum(w, axis=1, keepdims=True)
            vh = vt_ref[b * HL + h].astype(jnp.bfloat16)
            ctx = jnp.dot(w.astype(jnp.bfloat16), vh,
                          preferred_element_type=jnp.float32)
            woh = wo_ref[h * DH:(h + 1) * DH, :].astype(jnp.bfloat16)
            pb = jnp.dot(ctx.astype(jnp.bfloat16), woh,
                         preferred_element_type=jnp.float32)
            acc = pb if acc is None else acc + pb
        p_ref[b * SQ:(b + 1) * SQ, :] = acc
        pb_ref[b * SQ:(b + 1) * SQ, :] = acc.astype(jnp.bfloat16)

        for j in range(b * SQ // C, (b + 1) * SQ // C):
            @pl.when(j != my)
            def _():
                rdma = pltpu.make_async_remote_copy(
                    src_ref=pb_ref.at[pl.ds(j * C, C), :],
                    dst_ref=rs_ref.at[pl.ds(my * C, C), :],
                    send_sem=s1.at[j],
                    recv_sem=r1.at[my],
                    device_id=j,
                    device_id_type=pl.DeviceIdType.LOGICAL,
                )
                rdma.start()

    rs_ref[pl.ds(my * C, C), :] = pb_ref[pl.ds(my * C, C), :]

    for j in range(N_DEV):
        @pl.when(j != my)
        def _():
            rd = pltpu.make_async_remote_copy(
                src_ref=pb_ref.at[pl.ds(0, C), :],
                dst_ref=rs_ref.at[pl.ds(j * C, C), :],
                send_sem=s1.at[j],
                recv_sem=r1.at[j],
                device_id=j,
                device_id_type=pl.DeviceIdType.LOGICAL,
            )
            rd.wait_recv()

    for j in range(N_DEV):
        @pl.when(j != my)
        def _():
            pltpu.make_async_remote_copy(
                src_ref=pb_ref.at[pl.ds(j * C, C), :],
                dst_ref=rs_ref.at[pl.ds(j * C, C), :],
                send_sem=s1.at[j],
                recv_sem=r1.at[j],
                device_id=j,
                device_id_type=pl.DeviceIdType.LOGICAL,
            ).wait_send()

    red = rs_ref[0:C, :].astype(jnp.float32)
    for j in range(1, N_DEV):
        red = red + rs_ref[j * C:(j + 1) * C, :].astype(jnp.float32)
    red_ref[:, :] = red.astype(jnp.bfloat16)
    ag_ref[pl.ds(my * C, C), :] = red_ref[:, :]

    for k in range(N_DEV):
        @pl.when(k != my)
        def _():
            rdma = pltpu.make_async_remote_copy(
                src_ref=red_ref.at[:, :],
                dst_ref=ag_ref.at[pl.ds(my * C, C), :],
                send_sem=s2.at[k],
                recv_sem=r2.at[my],
                device_id=k,
                device_id_type=pl.DeviceIdType.LOGICAL,
            )
            rdma.start()

    for k in range(N_DEV):
        @pl.when(k != my)
        def _():
            rd = pltpu.make_async_remote_copy(
                src_ref=red_ref.at[:, :],
                dst_ref=ag_ref.at[pl.ds(k * C, C), :],
                send_sem=s2.at[k],
                recv_sem=r2.at[k],
                device_id=k,
                device_id_type=pl.DeviceIdType.LOGICAL,
            )
            rd.wait_recv()

    out_ref[:, :] = ag_ref[:, :].astype(jnp.float32)

    for k in range(N_DEV):
        @pl.when(k != my)
        def _():
            pltpu.make_async_remote_copy(
                src_ref=red_ref.at[:, :],
                dst_ref=ag_ref.at[pl.ds(my * C, C), :],
                send_sem=s2.at[k],
                recv_sem=r2.at[k],
                device_id=k,
                device_id_type=pl.DeviceIdType.LOGICAL,
            ).wait_send()


def kernel(x, Wq, K_ext, V_ext, Wo):
    out = pl.pallas_call(
        _body,
        out_shape=jax.ShapeDtypeStruct((ROWS, DMODEL), jnp.float32),
        in_specs=[
            pl.BlockSpec(memory_space=pltpu.VMEM),
            pl.BlockSpec(memory_space=pltpu.VMEM),
            pl.BlockSpec(memory_space=pl.ANY),
            pl.BlockSpec(memory_space=pl.ANY),
            pl.BlockSpec(memory_space=pltpu.VMEM),
        ],
        out_specs=pl.BlockSpec(memory_space=pltpu.VMEM),
        scratch_shapes=[
            pltpu.VMEM((B, SKV, HL, DH), jnp.float32),
            pltpu.VMEM((B, SKV, HL, DH), jnp.float32),
            pltpu.VMEM((B * HL, SKV, DH), jnp.float32),
            pltpu.VMEM((B * HL, SKV, DH), jnp.float32),
            pltpu.VMEM((ROWS, DMODEL), jnp.float32),
            pltpu.VMEM((ROWS, DMODEL), jnp.bfloat16),
            pltpu.VMEM((ROWS, DMODEL), jnp.bfloat16),
            pltpu.VMEM((C, DMODEL), jnp.bfloat16),
            pltpu.VMEM((ROWS, DMODEL), jnp.bfloat16),
            pltpu.SemaphoreType.DMA((2 * B,)),
            pltpu.SemaphoreType.DMA((2 * B * HL,)),
            pltpu.SemaphoreType.DMA((N_DEV,)),
            pltpu.SemaphoreType.DMA((N_DEV,)),
            pltpu.SemaphoreType.DMA((N_DEV,)),
            pltpu.SemaphoreType.DMA((N_DEV,)),
        ],
        compiler_params=pltpu.CompilerParams(collective_id=0),
    )(x.reshape(ROWS, DMODEL), Wq, K_ext, V_ext, Wo)
    return out.reshape(B, SQ, DMODEL)
